# baseline (speedup 1.0000x reference)
import functools

import jax
import jax.numpy as jnp
from jax import lax
from jax.experimental import pallas as pl
from jax.experimental.pallas import tpu as pltpu
from jax.experimental.pallas import tpu_sc as plsc

VOCAB = 49408
HIDDEN = 768
SEQ = 77
BATCH = 1024

NC = 2
NS = 16
LANES = 16
NW = NC * NS
BPW = BATCH // NW
KV = HIDDEN // LANES
CPS = 2
CH = BPW // CPS
NSTEP = SEQ * CPS
NSLOT = 7
K = 3


def _body(ids_hbm, table_hbm, pos_hbm, flags_hbm, out_hbm, ids_v, flags_v,
          *scratch):
  rows = list(scratch[0:NSLOT])
  prow = list(scratch[NSLOT:NSLOT + K])
  gsem = list(scratch[NSLOT + K:2 * NSLOT + K])
  ssem = list(scratch[2 * NSLOT + K:3 * NSLOT + K])
  psem = list(scratch[3 * NSLOT + K:3 * NSLOT + 2 * K])

  wid = lax.axis_index("s") * NC + lax.axis_index("c")

  pltpu.sync_copy(ids_hbm.at[wid], ids_v)
  pltpu.sync_copy(flags_hbm, flags_v)

  def flag(c):
    return flags_v[c // CPS, pl.ds(0, LANES)][0]

  def launch(c, b):
    pltpu.async_copy(table_hbm.at[ids_v.at[c]], rows[b], gsem[b])
    @pl.when(flag(c) != 0)
    def _():
      pltpu.async_copy(
          pos_hbm.at[pl.ds((c // CPS) * HIDDEN, HIDDEN)],
          prow[b % K], psem[b % K])

  def sc_dst(c):
    base = (c // CPS) * BATCH + wid * BPW + (c % CPS) * CH
    return out_hbm.at[pl.ds(base, CH)]

  def add_pos(c, b):
    @pl.when(flag(c) != 0)
    def _():
      pltpu.make_async_copy(
          pos_hbm.at[pl.ds((c // CPS) * HIDDEN, HIDDEN)], prow[b % K],
          psem[b % K]).wait()
      def kbody(k, _):
        pv = prow[b % K][pl.ds(k * LANES, LANES)]
        def jbody(j, _):
          r = rows[b][j, pl.ds(k * LANES, LANES)]
          rows[b][j, pl.ds(k * LANES, LANES)] = r + pv
          return 0
        return lax.fori_loop(0, CH, jbody, 0, unroll=8)
      lax.fori_loop(0, KV, kbody, 0)

  def visit(c, b):
    b2 = (b - K) % NSLOT
    pltpu.make_async_copy(table_hbm.at[ids_v.at[c]], rows[b], gsem[b]).wait()
    add_pos(c, b)
    pltpu.async_copy(rows[b], sc_dst(c), ssem[b])
    @pl.when(c >= K)
    def _():
      pltpu.make_async_copy(rows[b2], sc_dst(c - K), ssem[b2]).wait()
    @pl.when(c + NSLOT - K < NSTEP)
    def _():
      launch(c + NSLOT - K, b2)

  for c0 in range(NSLOT - K):
    launch(c0, c0)

  def loop_body(t, _):
    for b in range(NSLOT):
      visit(NSLOT * t + b, b)
    return 0
  lax.fori_loop(0, NSTEP // NSLOT, loop_body, 0)
  for c0 in range(NSLOT * (NSTEP // NSLOT), NSTEP):
    visit(c0, c0 % NSLOT)

  for c0 in range(NSTEP - K, NSTEP):
    pltpu.make_async_copy(rows[c0 % NSLOT], sc_dst(c0), ssem[c0 % NSLOT]).wait()


@functools.partial(jax.jit, donate_argnums=())
def _embed(ids_w, table, pos, flags):
  mesh = plsc.VectorSubcoreMesh(
      core_axis_name="c", subcore_axis_name="s",
      num_cores=NC, num_subcores=NS)
  run = pl.kernel(
      _body,
      out_type=jax.ShapeDtypeStruct((BATCH * SEQ, HIDDEN), jnp.float32),
      mesh=mesh,
      scratch_types=(
          [pltpu.VMEM((NSTEP, CH), jnp.int32)]
          + [pltpu.VMEM((SEQ, LANES), jnp.int32)]
          + [pltpu.VMEM((CH, HIDDEN), jnp.float32)] * NSLOT
          + [pltpu.VMEM((HIDDEN,), jnp.float32)] * K
          + [pltpu.SemaphoreType.DMA] * (2 * NSLOT + K)
      ),
  )
  return run(ids_w, table, pos, flags)


def kernel(input_ids, token_embedding, position_embedding):
  ids32 = input_ids.astype(jnp.int32)
  ids_w = ids32.reshape(NW, BPW, SEQ).transpose(0, 2, 1).reshape(NW, NSTEP, CH)
  pos_flat = position_embedding.reshape(SEQ * HIDDEN)
  pos_bits = position_embedding.view(jnp.int32) & jnp.int32(0x7FFFFFFF)
  flags = jnp.any(pos_bits != 0, axis=1).astype(jnp.int32)
  flags = jnp.broadcast_to(flags[:, None], (SEQ, LANES))
  out = _embed(ids_w, token_embedding, pos_flat, flags)
  return out.reshape(SEQ, BATCH, HIDDEN).transpose(1, 0, 2)


# speedup vs baseline: 1.0033x; 1.0033x over previous
import functools

import jax
import jax.numpy as jnp
from jax import lax
from jax.experimental import pallas as pl
from jax.experimental.pallas import tpu as pltpu
from jax.experimental.pallas import tpu_sc as plsc

VOCAB = 49408
HIDDEN = 768
SEQ = 77
BATCH = 1024

NC = 2
NS = 16
LANES = 16
NW = NC * NS
BPW = BATCH // NW
KV = HIDDEN // LANES
GSPLIT = 2
CH = BPW // GSPLIT
NSTEP = SEQ
NSLOT = 4
K = 2


def _body(ids_hbm, table_hbm, pos_hbm, flags_hbm, out_hbm, ids_v, flags_v,
          *scratch):
  rows = list(scratch[0:NSLOT])
  prow = list(scratch[NSLOT:NSLOT + K])
  gsem = list(scratch[NSLOT + K:NSLOT + K + GSPLIT * NSLOT])
  ssem = list(scratch[NSLOT + K + GSPLIT * NSLOT:
                      NSLOT + K + GSPLIT * NSLOT + NSLOT])
  psem = list(scratch[NSLOT + K + GSPLIT * NSLOT + NSLOT:
                      NSLOT + K + GSPLIT * NSLOT + NSLOT + K])

  wid = lax.axis_index("s") * NC + lax.axis_index("c")

  pltpu.sync_copy(ids_hbm.at[wid], ids_v)
  pltpu.sync_copy(flags_hbm, flags_v)

  def flag(c):
    return flags_v[c, pl.ds(0, LANES)][0]

  def launch(c, b):
    for g in range(GSPLIT):
      pltpu.async_copy(table_hbm.at[ids_v.at[GSPLIT * c + g]],
                       rows[b].at[pl.ds(g * CH, CH)],
                       gsem[GSPLIT * b + g])
    @pl.when(flag(c) != 0)
    def _():
      pltpu.async_copy(
          pos_hbm.at[pl.ds(c * HIDDEN, HIDDEN)],
          prow[b % K], psem[b % K])

  def sc_dst(c):
    return out_hbm.at[pl.ds(c * BATCH + wid * BPW, BPW)]

  def add_pos(c, b):
    @pl.when(flag(c) != 0)
    def _():
      pltpu.make_async_copy(
          pos_hbm.at[pl.ds(c * HIDDEN, HIDDEN)], prow[b % K],
          psem[b % K]).wait()
      def kbody(k, _):
        pv = prow[b % K][pl.ds(k * LANES, LANES)]
        def jbody(j, _):
          r = rows[b][j, pl.ds(k * LANES, LANES)]
          rows[b][j, pl.ds(k * LANES, LANES)] = r + pv
          return 0
        return lax.fori_loop(0, BPW, jbody, 0, unroll=8)
      lax.fori_loop(0, KV, kbody, 0)

  def visit(c, b):
    b2 = (b - K) % NSLOT
    for g in range(GSPLIT):
      pltpu.make_async_copy(table_hbm.at[ids_v.at[GSPLIT * c + g]],
                            rows[b].at[pl.ds(g * CH, CH)],
                            gsem[GSPLIT * b + g]).wait()
    add_pos(c, b)
    pltpu.async_copy(rows[b], sc_dst(c), ssem[b])
    @pl.when(c >= K)
    def _():
      pltpu.make_async_copy(rows[b2], sc_dst(c - K), ssem[b2]).wait()
    @pl.when(c + NSLOT - K < NSTEP)
    def _():
      launch(c + NSLOT - K, b2)

  for c0 in range(NSLOT - K):
    launch(c0, c0)

  def loop_body(t, _):
    for b in range(NSLOT):
      visit(NSLOT * t + b, b)
    return 0
  lax.fori_loop(0, NSTEP // NSLOT, loop_body, 0)
  for c0 in range(NSLOT * (NSTEP // NSLOT), NSTEP):
    visit(c0, c0 % NSLOT)

  for c0 in range(NSTEP - K, NSTEP):
    pltpu.make_async_copy(rows[c0 % NSLOT], sc_dst(c0), ssem[c0 % NSLOT]).wait()


@functools.partial(jax.jit, donate_argnums=())
def _embed(ids_w, table, pos, flags):
  mesh = plsc.VectorSubcoreMesh(
      core_axis_name="c", subcore_axis_name="s",
      num_cores=NC, num_subcores=NS)
  run = pl.kernel(
      _body,
      out_type=jax.ShapeDtypeStruct((BATCH * SEQ, HIDDEN), jnp.float32),
      mesh=mesh,
      scratch_types=(
          [pltpu.VMEM((SEQ * GSPLIT, CH), jnp.int32)]
          + [pltpu.VMEM((SEQ, LANES), jnp.int32)]
          + [pltpu.VMEM((BPW, HIDDEN), jnp.float32)] * NSLOT
          + [pltpu.VMEM((HIDDEN,), jnp.float32)] * K
          + [pltpu.SemaphoreType.DMA] * ((GSPLIT + 1) * NSLOT + K)
      ),
  )
  return run(ids_w, table, pos, flags)


def kernel(input_ids, token_embedding, position_embedding):
  ids32 = input_ids.astype(jnp.int32)
  ids_w = ids32.reshape(NW, BPW, SEQ).transpose(0, 2, 1).reshape(
      NW, SEQ * GSPLIT, CH)
  pos_flat = position_embedding.reshape(SEQ * HIDDEN)
  pos_bits = position_embedding.view(jnp.int32) & jnp.int32(0x7FFFFFFF)
  flags = jnp.any(pos_bits != 0, axis=1).astype(jnp.int32)
  flags = jnp.broadcast_to(flags[:, None], (SEQ, LANES))
  out = _embed(ids_w, token_embedding, pos_flat, flags)
  return out.reshape(SEQ, BATCH, HIDDEN).transpose(1, 0, 2)


# speedup vs baseline: 1.0101x; 1.0068x over previous
import functools

import jax
import jax.numpy as jnp
from jax import lax
from jax.experimental import pallas as pl
from jax.experimental.pallas import tpu as pltpu
from jax.experimental.pallas import tpu_sc as plsc

VOCAB = 49408
HIDDEN = 768
SEQ = 77
BATCH = 1024

NC = 2
NS = 16
LANES = 16
NW = NC * NS
BPW = BATCH // NW
KV = HIDDEN // LANES
GSPLIT = 1
CH = BPW // GSPLIT
NSTEP = SEQ
NSLOT = 4
K = 2


def _body(ids_hbm, table_hbm, pos_hbm, flags_hbm, out_hbm, ids_v, flags_v,
          *scratch):
  rows = list(scratch[0:NSLOT])
  prow = list(scratch[NSLOT:NSLOT + K])
  gsem = list(scratch[NSLOT + K:NSLOT + K + GSPLIT * NSLOT])
  ssem = list(scratch[NSLOT + K + GSPLIT * NSLOT:
                      NSLOT + K + GSPLIT * NSLOT + NSLOT])
  psem = list(scratch[NSLOT + K + GSPLIT * NSLOT + NSLOT:
                      NSLOT + K + GSPLIT * NSLOT + NSLOT + K])

  wid = lax.axis_index("s") * NC + lax.axis_index("c")

  pltpu.sync_copy(ids_hbm.at[wid], ids_v)
  pltpu.sync_copy(flags_hbm, flags_v)

  def flag(c):
    return flags_v[c, pl.ds(0, LANES)][0]

  def launch(c, b):
    for g in range(GSPLIT):
      pltpu.async_copy(table_hbm.at[ids_v.at[GSPLIT * c + g]],
                       rows[b].at[pl.ds(g * CH, CH)],
                       gsem[GSPLIT * b + g])
    @pl.when(flag(c) != 0)
    def _():
      pltpu.async_copy(
          pos_hbm.at[pl.ds(c * HIDDEN, HIDDEN)],
          prow[b % K], psem[b % K])

  def sc_dst(c):
    return out_hbm.at[pl.ds(c * BATCH + wid * BPW, BPW)]

  def add_pos(c, b):
    @pl.when(flag(c) != 0)
    def _():
      pltpu.make_async_copy(
          pos_hbm.at[pl.ds(c * HIDDEN, HIDDEN)], prow[b % K],
          psem[b % K]).wait()
      def kbody(k, _):
        pv = prow[b % K][pl.ds(k * LANES, LANES)]
        def jbody(j, _):
          r = rows[b][j, pl.ds(k * LANES, LANES)]
          rows[b][j, pl.ds(k * LANES, LANES)] = r + pv
          return 0
        return lax.fori_loop(0, BPW, jbody, 0, unroll=8)
      lax.fori_loop(0, KV, kbody, 0)

  def visit(c, b):
    b2 = (b - K) % NSLOT
    for g in range(GSPLIT):
      pltpu.make_async_copy(table_hbm.at[ids_v.at[GSPLIT * c + g]],
                            rows[b].at[pl.ds(g * CH, CH)],
                            gsem[GSPLIT * b + g]).wait()
    add_pos(c, b)
    pltpu.async_copy(rows[b], sc_dst(c), ssem[b])
    @pl.when(c >= K)
    def _():
      pltpu.make_async_copy(rows[b2], sc_dst(c - K), ssem[b2]).wait()
    @pl.when(c + NSLOT - K < NSTEP)
    def _():
      launch(c + NSLOT - K, b2)

  for c0 in range(NSLOT - K):
    launch(c0, c0)

  def loop_body(t, _):
    for b in range(NSLOT):
      visit(NSLOT * t + b, b)
    return 0
  lax.fori_loop(0, NSTEP // NSLOT, loop_body, 0)
  for c0 in range(NSLOT * (NSTEP // NSLOT), NSTEP):
    visit(c0, c0 % NSLOT)

  for c0 in range(NSTEP - K, NSTEP):
    pltpu.make_async_copy(rows[c0 % NSLOT], sc_dst(c0), ssem[c0 % NSLOT]).wait()


@functools.partial(jax.jit, donate_argnums=())
def _embed(ids_w, table, pos, flags):
  mesh = plsc.VectorSubcoreMesh(
      core_axis_name="c", subcore_axis_name="s",
      num_cores=NC, num_subcores=NS)
  run = pl.kernel(
      _body,
      out_type=jax.ShapeDtypeStruct((BATCH * SEQ, HIDDEN), jnp.float32),
      mesh=mesh,
      scratch_types=(
          [pltpu.VMEM((SEQ * GSPLIT, CH), jnp.int32)]
          + [pltpu.VMEM((SEQ, LANES), jnp.int32)]
          + [pltpu.VMEM((BPW, HIDDEN), jnp.float32)] * NSLOT
          + [pltpu.VMEM((HIDDEN,), jnp.float32)] * K
          + [pltpu.SemaphoreType.DMA] * ((GSPLIT + 1) * NSLOT + K)
      ),
  )
  return run(ids_w, table, pos, flags)


def kernel(input_ids, token_embedding, position_embedding):
  ids32 = input_ids.astype(jnp.int32)
  ids_w = ids32.reshape(NW, BPW, SEQ).transpose(0, 2, 1).reshape(
      NW, SEQ * GSPLIT, CH)
  pos_flat = position_embedding.reshape(SEQ * HIDDEN)
  pos_bits = position_embedding.view(jnp.int32) & jnp.int32(0x7FFFFFFF)
  flags = jnp.any(pos_bits != 0, axis=1).astype(jnp.int32)
  flags = jnp.broadcast_to(flags[:, None], (SEQ, LANES))
  out = _embed(ids_w, token_embedding, pos_flat, flags)
  return out.reshape(SEQ, BATCH, HIDDEN).transpose(1, 0, 2)
